# baseline (speedup 1.0000x reference)
import functools

import jax
import jax.numpy as jnp
from jax import lax
from jax.experimental import pallas as pl
from jax.experimental.pallas import tpu as pltpu
from jax.experimental.pallas import tpu_sc as plsc

N_NODES = 10000
N_EDGES = 320000
NUM_GAUSSIANS = 128
NUM_FILTERS = 128
CUTOFF = 5.0
GAUSSIAN_WIDTH = CUTOFF / (NUM_GAUSSIANS - 1)

T = 512
TROWS = 640
SCALE = T / CUTOFF
TPK = (T + 2) * 64

NC, NS = 2, 16
NW = NC * NS
C = 64
NCHUNK = N_EDGES // C
NITER = -(-NCHUNK // NW)


def _ssp(x):
    return jnp.maximum(x, 0.0) + jnp.log(1.0 + jnp.exp(-jnp.abs(x))) - 0.6931471805599453


def _table_body(w1_ref, b1_ref, w2_ref, b2_ref, o_ref):
    i = pl.program_id(0)
    rows = lax.broadcasted_iota(jnp.int32, (128, NUM_GAUSSIANS), 0).astype(jnp.float32)
    cols = lax.broadcasted_iota(jnp.int32, (128, NUM_GAUSSIANS), 1).astype(jnp.float32)
    r = (rows + jnp.float32(i) * 128.0) * (CUTOFF / T)
    c = cols * GAUSSIAN_WIDTH
    g = jnp.exp(-((r - c) ** 2) / (2.0 * GAUSSIAN_WIDTH * GAUSSIAN_WIDTH))
    y = _ssp(jnp.dot(g, w1_ref[...], preferred_element_type=jnp.float32) + b1_ref[...])
    w = _ssp(jnp.dot(y, w2_ref[...], preferred_element_type=jnp.float32) + b2_ref[...])
    cut = jnp.where(r < CUTOFF, 0.5 * jnp.cos((jnp.pi / CUTOFF) * r) + 0.5, 0.0)
    o_ref[...] = w * cut


_build_table = pl.pallas_call(
    _table_body,
    grid=(TROWS // 128,),
    in_specs=[
        pl.BlockSpec((NUM_GAUSSIANS, NUM_FILTERS), lambda i: (0, 0)),
        pl.BlockSpec((1, NUM_FILTERS), lambda i: (0, 0)),
        pl.BlockSpec((NUM_FILTERS, NUM_FILTERS), lambda i: (0, 0)),
        pl.BlockSpec((1, NUM_FILTERS), lambda i: (0, 0)),
    ],
    out_specs=pl.BlockSpec((128, NUM_FILTERS), lambda i: (i, 0)),
    out_shape=jax.ShapeDtypeStruct((TROWS, NUM_FILTERS), jnp.float32),
)


def _idx_body(src_hbm, dst_hbm, px_hbm, py_hbm, pz_hbm,
              idx_hbm, frac_hbm,
              px_v, py_v, pz_v, srcb, dstb, idxb, fracb):
    cid = lax.axis_index("c")
    sid = lax.axis_index("s")
    wid = cid * NS + sid

    pltpu.sync_copy(px_hbm, px_v)
    pltpu.sync_copy(py_hbm, py_v)
    pltpu.sync_copy(pz_hbm, pz_v)

    @pl.loop(0, NITER)
    def _p(j):
        ch = j * NW + wid

        @pl.when(ch < NCHUNK)
        def _chunk():
            base = ch * C
            pltpu.sync_copy(src_hbm.at[pl.ds(base, C)], srcb)
            pltpu.sync_copy(dst_hbm.at[pl.ds(base, C)], dstb)
            for k in range(C // 16):
                off = k * 16
                sv = srcb[pl.ds(off, 16)]
                dv = dstb[pl.ds(off, 16)]
                dx = plsc.load_gather(px_v, [sv]) - plsc.load_gather(px_v, [dv])
                dy = plsc.load_gather(py_v, [sv]) - plsc.load_gather(py_v, [dv])
                dz = plsc.load_gather(pz_v, [sv]) - plsc.load_gather(pz_v, [dv])
                rsq = dx * dx + dy * dy + dz * dz + 1e-12
                ii = jnp.int32(0x5F3759DF) - (plsc.bitcast(rsq, jnp.int32) >> 1)
                yv = plsc.bitcast(ii, jnp.float32)
                yv = yv * (1.5 - 0.5 * rsq * yv * yv)
                yv = yv * (1.5 - 0.5 * rsq * yv * yv)
                rr = rsq * yv
                t_ = rr * SCALE
                idx = jnp.minimum(t_.astype(jnp.int32), T)
                idxb[pl.ds(off, 16)] = idx
                fracb[pl.ds(off, 16)] = t_ - idx.astype(jnp.float32)
            pltpu.sync_copy(idxb, idx_hbm.at[pl.ds(base, C)])
            pltpu.sync_copy(fracb, frac_hbm.at[pl.ds(base, C)])


_idx_kernel = functools.partial(
    pl.kernel,
    out_type=(jax.ShapeDtypeStruct((N_EDGES,), jnp.int32),
              jax.ShapeDtypeStruct((N_EDGES,), jnp.float32)),
    mesh=plsc.VectorSubcoreMesh(core_axis_name="c", subcore_axis_name="s"),
    compiler_params=pltpu.CompilerParams(needs_layout_passes=False),
    scratch_types=[
        pltpu.VMEM((N_NODES,), jnp.float32),
        pltpu.VMEM((N_NODES,), jnp.float32),
        pltpu.VMEM((N_NODES,), jnp.float32),
        pltpu.VMEM((C,), jnp.int32),
        pltpu.VMEM((C,), jnp.int32),
        pltpu.VMEM((C,), jnp.int32),
        pltpu.VMEM((C,), jnp.float32),
    ],
)(_idx_body)


def _edge_body(src_hbm, dst_hbm, idx_hbm, frac_hbm, tpk_hbm, feat_hbm,
               out_hbm,
               tpk_v, srcb, dstb, tblb, fracb, inp_rows,
               shared, sem_a):
    cid = lax.axis_index("c")
    sid = lax.axis_index("s")
    wid = cid * NS + sid

    pltpu.sync_copy(tpk_hbm, tpk_v)

    @pl.loop(0, C)
    def _zb(e):
        for k in range(8):
            inp_rows[e, pl.ds(k * 16, 16)] = jnp.zeros((16,), jnp.float32)

    nz = N_NODES // C
    for t in range(-(-nz // NS)):
        ch = sid + NS * t

        @pl.when(ch < nz)
        def _z():
            pltpu.sync_copy(inp_rows, shared.at[pl.ds(ch * C, C)])

    @pl.when(sid == 0)
    def _ztail():
        pltpu.sync_copy(inp_rows.at[pl.ds(0, N_NODES - nz * C)],
                        shared.at[pl.ds(nz * C, N_NODES - nz * C)])

    plsc.subcore_barrier()

    @pl.loop(0, NITER)
    def _p(j):
        ch = j * NW + wid

        @pl.when(ch < NCHUNK)
        def _chunk():
            base = ch * C
            pltpu.sync_copy(src_hbm.at[pl.ds(base, C)], srcb)
            cp1 = pltpu.async_copy(feat_hbm.at[srcb], inp_rows, sem_a)
            pltpu.sync_copy(dst_hbm.at[pl.ds(base, C)], dstb)
            pltpu.sync_copy(idx_hbm.at[pl.ds(base, C)], tblb)
            pltpu.sync_copy(frac_hbm.at[pl.ds(base, C)], fracb)
            cp1.wait()

            lanes = lax.iota(jnp.int32, 16)
            halfw = lanes >> 1
            shv = (lanes & 1) * 16

            @pl.loop(0, C)
            def _mul(e):
                g = (e >> 4) << 4
                l = e & 15
                m = lanes == l
                idxv = tblb[pl.ds(g, 16)]
                frv = fracb[pl.ds(g, 16)]
                idx_e = jnp.sum(jnp.where(m, idxv, 0))
                fr = jnp.broadcast_to(jnp.sum(jnp.where(m, frv, 0.0)), (16,))
                rowbase = jnp.broadcast_to(idx_e * 64, (16,)) + halfw
                for k in range(8):
                    iv = rowbase + 8 * k
                    wlo = plsc.load_gather(tpk_v, [iv])
                    whi = plsc.load_gather(tpk_v, [iv + 64])
                    lo = plsc.bitcast(
                        lax.shift_left(lax.shift_right_logical(wlo, shv), 16),
                        jnp.float32)
                    hi = plsc.bitcast(
                        lax.shift_left(lax.shift_right_logical(whi, shv), 16),
                        jnp.float32)
                    sl = pl.ds(k * 16, 16)
                    inp_rows[e, sl] = inp_rows[e, sl] * (lo + fr * (hi - lo))

            pltpu.sync_copy(inp_rows, shared.at[dstb], add=True)

    plsc.subcore_barrier()

    @pl.when(sid == 0)
    def _dump():
        pltpu.sync_copy(shared, out_hbm.at[cid])


_edge_kernel = functools.partial(
    pl.kernel,
    out_type=jax.ShapeDtypeStruct((NC, N_NODES, NUM_FILTERS), jnp.float32),
    mesh=plsc.VectorSubcoreMesh(core_axis_name="c", subcore_axis_name="s"),
    compiler_params=pltpu.CompilerParams(needs_layout_passes=False),
    scratch_types=[
        pltpu.VMEM((TPK,), jnp.int32),
        pltpu.VMEM((C,), jnp.int32),
        pltpu.VMEM((C,), jnp.int32),
        pltpu.VMEM((C,), jnp.int32),
        pltpu.VMEM((C,), jnp.float32),
        pltpu.VMEM((C, NUM_FILTERS), jnp.float32),
        pltpu.VMEM_SHARED((N_NODES, NUM_FILTERS), jnp.float32),
        pltpu.SemaphoreType.DMA,
    ],
)(_edge_body)


def _sum_body(p_ref, o_ref):
    o_ref[...] = p_ref[0] + p_ref[1]


_sum_parts = pl.pallas_call(
    _sum_body,
    grid=(5,),
    in_specs=[pl.BlockSpec((2, 2000, NUM_FILTERS), lambda i: (0, i, 0))],
    out_specs=pl.BlockSpec((2000, NUM_FILTERS), lambda i: (i, 0)),
    out_shape=jax.ShapeDtypeStruct((N_NODES, NUM_FILTERS), jnp.float32),
)


def kernel(positions, input, edge_index, weights1, biases1, weights2, biases2):
    src = edge_index[0]
    dst = edge_index[1]
    px = positions[:, 0]
    py = positions[:, 1]
    pz = positions[:, 2]
    tab = _build_table(weights1, biases1.reshape(1, -1),
                       weights2, biases2.reshape(1, -1))
    tpk = tab[:T + 2].astype(jnp.bfloat16).view(jnp.int32).reshape(-1)
    eidx, efrac = _idx_kernel(src, dst, px, py, pz)
    parts = _edge_kernel(src, dst, eidx, efrac, tpk, input)
    return _sum_parts(parts)
